# baseline (speedup 1.0000x reference)
import jax
import jax.numpy as jnp
from jax import lax
from jax.experimental import pallas as pl
from jax.experimental.pallas import tpu as pltpu
from jax.experimental.pallas import tpu_sc as plsc

S = 2048
D = 1024
NW = 32
RPW = S // NW
CRK = 8
NCH = RPW // CRK
L = 16


def _sc_body(bos_hbm, idx_hbm, embeds_hbm, speech_hbm, pos_hbm, out_hbm,
             buf0, buf1, buf2, buf3, buf4, buf5, tokbuf, ixbuf, row_s, row_p, row_i,
             rsem, wsem, gsem_t, gsem_i, gsem_s, gsem_p):
    cid = lax.axis_index("c")
    sid = lax.axis_index("s")
    wid = sid * 2 + cid
    r0 = wid * RPW
    bufs = (buf0, buf1, buf2, buf3, buf4, buf5)
    is_bos = sid == 0

    @pl.when(is_bos)
    def _bos_fetch():
        pltpu.async_copy(bos_hbm, tokbuf, gsem_t)
        pltpu.async_copy(idx_hbm, ixbuf, gsem_i)

    def rd(c, k, b):
        return pltpu.async_copy(
            embeds_hbm.at[b, pl.ds(r0 + c * CRK, CRK), :],
            bufs[k].at[:, b, :], rsem.at[k, b])

    def wr(c, k):
        return pltpu.async_copy(
            bufs[k], out_hbm.at[pl.ds(r0 + c * CRK, CRK), :, :], wsem.at[k])

    NBUF = 6
    reads = {}
    for c in range(NBUF):
        for b in (0, 1):
            reads[(c, b)] = rd(c, c, b)

    @pl.when(is_bos)
    def _bos_gather():
        pltpu.make_async_copy(bos_hbm, tokbuf, gsem_t).wait()
        pltpu.make_async_copy(idx_hbm, ixbuf, gsem_i).wait()
        pltpu.async_copy(speech_hbm.at[tokbuf], row_s, gsem_s)
        pltpu.async_copy(pos_hbm.at[ixbuf], row_p, gsem_p)

    writes = [None] * NCH
    for c in range(NCH):
        k = c % NBUF
        if c >= NBUF:
            writes[c - NBUF].wait()
            for b in (0, 1):
                reads[(c, b)] = rd(c, k, b)
        reads[(c, 0)].wait()
        reads[(c, 1)].wait()
        writes[c] = wr(c, k)

    @pl.when(is_bos)
    def _bos_write():
        pltpu.make_async_copy(speech_hbm.at[tokbuf], row_s, gsem_s).wait()
        pltpu.make_async_copy(pos_hbm.at[ixbuf], row_p, gsem_p).wait()
        for i in range(D // L):
            sl = pl.ds(i * L, L)
            row_i[0, 0, sl] = row_s[0, sl] + row_p[0, sl]
        pltpu.sync_copy(
            row_i, out_hbm.at[pl.ds(S, 1), pl.ds(cid, 1), :])

    for c in range(max(0, NCH - NBUF), NCH):
        writes[c].wait()


def kernel(bos_token, embeds, idx, speech_emb, pos_emb):
    mesh = plsc.VectorSubcoreMesh(core_axis_name="c", subcore_axis_name="s")
    sc_call = pl.kernel(
        _sc_body,
        mesh=mesh,
        out_type=jax.ShapeDtypeStruct((S + 1, 2, D), jnp.float32),
        scratch_types=[
            pltpu.VMEM((CRK, 2, D), jnp.float32),
            pltpu.VMEM((CRK, 2, D), jnp.float32),
            pltpu.VMEM((CRK, 2, D), jnp.float32),
            pltpu.VMEM((CRK, 2, D), jnp.float32),
            pltpu.VMEM((CRK, 2, D), jnp.float32),
            pltpu.VMEM((CRK, 2, D), jnp.float32),
            pltpu.VMEM((1,), jnp.int32),
            pltpu.VMEM((1,), jnp.int32),
            pltpu.VMEM((1, D), jnp.float32),
            pltpu.VMEM((1, D), jnp.float32),
            pltpu.VMEM((1, 1, D), jnp.float32),
            pltpu.SemaphoreType.DMA((6, 2)),
            pltpu.SemaphoreType.DMA((6,)),
            pltpu.SemaphoreType.DMA,
            pltpu.SemaphoreType.DMA,
            pltpu.SemaphoreType.DMA,
            pltpu.SemaphoreType.DMA,
        ],
    )
    out_t = sc_call(bos_token.reshape(1), idx, embeds, speech_emb, pos_emb)
    return jnp.transpose(out_t, (1, 0, 2))


# speedup vs baseline: 1.0321x; 1.0321x over previous
import jax
import jax.numpy as jnp
from jax import lax
from jax.experimental import pallas as pl
from jax.experimental.pallas import tpu as pltpu
from jax.experimental.pallas import tpu_sc as plsc

S = 2048
D = 1024
NW = 32
RPW = S // NW
CRK = 16
NCH = RPW // CRK
L = 16


def _sc_body(bos_hbm, idx_hbm, embeds_hbm, speech_hbm, pos_hbm, out_hbm,
             buf0, buf1, buf2, tokbuf, ixbuf, row_s, row_p, row_i,
             rsem, wsem, gsem_t, gsem_i, gsem_s, gsem_p):
    cid = lax.axis_index("c")
    sid = lax.axis_index("s")
    wid = sid * 2 + cid
    r0 = wid * RPW
    bufs = (buf0, buf1, buf2)
    is_bos = sid == 0

    @pl.when(is_bos)
    def _bos_fetch():
        pltpu.async_copy(bos_hbm, tokbuf, gsem_t)
        pltpu.async_copy(idx_hbm, ixbuf, gsem_i)

    def rd(c, k, b):
        return pltpu.async_copy(
            embeds_hbm.at[b, pl.ds(r0 + c * CRK, CRK), :],
            bufs[k].at[b], rsem.at[k, b])

    def wr(c, k):
        w0 = pltpu.async_copy(
            bufs[k].at[0], out_hbm.at[pl.ds(r0 + c * CRK, CRK), 0, :],
            wsem.at[k, 0])
        w1 = pltpu.async_copy(
            bufs[k].at[1], out_hbm.at[pl.ds(r0 + c * CRK, CRK), 1, :],
            wsem.at[k, 1])
        return (w0, w1)

    reads = {}
    for c in (0, 1, 2):
        for b in (0, 1):
            reads[(c, b)] = rd(c, c, b)

    @pl.when(is_bos)
    def _bos_gather():
        pltpu.make_async_copy(bos_hbm, tokbuf, gsem_t).wait()
        pltpu.make_async_copy(idx_hbm, ixbuf, gsem_i).wait()
        pltpu.async_copy(speech_hbm.at[tokbuf], row_s, gsem_s)
        pltpu.async_copy(pos_hbm.at[ixbuf], row_p, gsem_p)

    writes = [None] * NCH
    for c in range(NCH):
        k = c % 3
        if c >= 3:
            writes[c - 3][0].wait()
            writes[c - 3][1].wait()
            for b in (0, 1):
                reads[(c, b)] = rd(c, k, b)
        reads[(c, 0)].wait()
        reads[(c, 1)].wait()
        writes[c] = wr(c, k)

    @pl.when(is_bos)
    def _bos_write():
        pltpu.make_async_copy(speech_hbm.at[tokbuf], row_s, gsem_s).wait()
        pltpu.make_async_copy(pos_hbm.at[ixbuf], row_p, gsem_p).wait()
        for i in range(D // L):
            sl = pl.ds(i * L, L)
            row_i[0, 0, sl] = row_s[0, sl] + row_p[0, sl]
        pltpu.sync_copy(
            row_i, out_hbm.at[pl.ds(S, 1), pl.ds(cid, 1), :])

    for c in range(NCH - 3, NCH):
        writes[c][0].wait()
        writes[c][1].wait()


def kernel(bos_token, embeds, idx, speech_emb, pos_emb):
    mesh = plsc.VectorSubcoreMesh(core_axis_name="c", subcore_axis_name="s")
    sc_call = pl.kernel(
        _sc_body,
        mesh=mesh,
        out_type=jax.ShapeDtypeStruct((S + 1, 2, D), jnp.float32),
        scratch_types=[
            pltpu.VMEM((2, CRK, D), jnp.float32),
            pltpu.VMEM((2, CRK, D), jnp.float32),
            pltpu.VMEM((2, CRK, D), jnp.float32),
            pltpu.VMEM((1,), jnp.int32),
            pltpu.VMEM((1,), jnp.int32),
            pltpu.VMEM((1, D), jnp.float32),
            pltpu.VMEM((1, D), jnp.float32),
            pltpu.VMEM((1, 1, D), jnp.float32),
            pltpu.SemaphoreType.DMA((3, 2)),
            pltpu.SemaphoreType.DMA((3, 2)),
            pltpu.SemaphoreType.DMA,
            pltpu.SemaphoreType.DMA,
            pltpu.SemaphoreType.DMA,
            pltpu.SemaphoreType.DMA,
        ],
    )
    out_t = sc_call(bos_token.reshape(1), idx, embeds, speech_emb, pos_emb)
    return jnp.transpose(out_t, (1, 0, 2))
